# Initial kernel scaffold; baseline (speedup 1.0000x reference)
import jax
import jax.numpy as jnp
from jax.experimental import pallas as pl


def kernel(x, edge_index, edge_attr, u, batch, W1a, b1a, W1b, b1b, W2a, b2a, W2b, b2b):
    raise NotImplementedError("write your pallas kernel here")


# speedup vs baseline: 1.3643x; 1.3643x over previous
import jax
import jax.numpy as jnp
from jax import lax
from jax.experimental import pallas as pl
from jax.experimental.pallas import tpu as pltpu
from jax.experimental.pallas import tpu_sc as plsc

N = 10000
E = 320000
N_F = 128
E_F = 16
HID = 128
U_F = 16
G = 64

NC, NS = 2, 16
FH = HID // NC
E_PER_TILE = E // NS
CB = 80
CHUNKS = E_PER_TILE // CB
NP = 10240
ZR = 128
N_PER_TILE = NP // NS


def _xw_body(x_ref, w_ref, o_ref):
    o_ref[...] = jnp.dot(x_ref[...], w_ref[0],
                         preferred_element_type=jnp.float32)


def _project_x(x, W1a_x):
    bn = 2000
    return pl.pallas_call(
        _xw_body,
        grid=(NC, N // bn),
        in_specs=[
            pl.BlockSpec((bn, N_F), lambda c, i: (i, 0)),
            pl.BlockSpec((1, N_F, FH), lambda c, i: (c, 0, 0)),
        ],
        out_specs=pl.BlockSpec((bn, FH), lambda c, i: (c * (N // bn) + i, 0)),
        out_shape=jax.ShapeDtypeStruct((NC * N, FH), jnp.float32),
    )(x, W1a_x.reshape(N_F, NC, FH).transpose(1, 0, 2))


def _ew_body(ea_ref, w_ref, b_ref, o_ref):
    o_ref[...] = jnp.dot(ea_ref[...], w_ref[0],
                         preferred_element_type=jnp.float32) + b_ref[0]


def _project_edges(edge_attr, W1a_e, b1a):
    be = 8000
    return pl.pallas_call(
        _ew_body,
        grid=(NC, E // be),
        in_specs=[
            pl.BlockSpec((be, E_F), lambda c, i: (i, 0)),
            pl.BlockSpec((1, E_F, FH), lambda c, i: (c, 0, 0)),
            pl.BlockSpec((1, 1, FH), lambda c, i: (c, 0, 0)),
        ],
        out_specs=pl.BlockSpec((be, FH), lambda c, i: (c * (E // be) + i, 0)),
        out_shape=jax.ShapeDtypeStruct((NC * E, FH), jnp.float32),
    )(edge_attr, W1a_e.reshape(E_F, NC, FH).transpose(1, 0, 2),
      b1a.reshape(NC, 1, FH))


def _edge_sc_body(row_h, col_h, ew_h, xw_h, out_h,
                  ridx, cidx, ew_v, g_v, o_v, z_v, acc_sh, sem):
    cid = lax.axis_index("c")
    sid = lax.axis_index("s")

    zvec = jnp.zeros((16,), jnp.float32)
    cvec = jnp.where(lax.iota(jnp.int32, 16) == 0, 1.0, 0.0)

    def zb(i, c):
        for j in range(HID // 16):
            z_v[i, pl.ds(j * 16, 16)] = zvec
        return c
    lax.fori_loop(0, ZR, zb, 0)
    for k in range(N_PER_TILE // ZR):
        pltpu.sync_copy(z_v, acc_sh.at[pl.ds(sid * N_PER_TILE + k * ZR, ZR)])

    def cb(i, c):
        o_v[i, pl.ds(FH, 16)] = cvec
        for j in range(FH // 16 + 1, HID // 16):
            o_v[i, pl.ds(j * 16, 16)] = zvec
        return c
    lax.fori_loop(0, CB, cb, 0)

    plsc.subcore_barrier()

    tile_base = sid * E_PER_TILE
    roff = cid * N
    eoff = cid * E

    def chunk(t, c):
        base = tile_base + t * CB
        pltpu.sync_copy(row_h.at[pl.ds(base, CB)], ridx)
        pltpu.sync_copy(col_h.at[pl.ds(base, CB)], cidx)
        pltpu.sync_copy(ew_h.at[pl.ds(eoff + base, CB)], ew_v)

        def ab(j, cc):
            s = pl.ds(j * 16, 16)
            ridx[s] = ridx[s] + roff
            return cc
        lax.fori_loop(0, CB // 16, ab, 0)
        pltpu.async_copy(xw_h.at[ridx], g_v, sem).wait()

        def eb(i, cc):
            for j in range(FH // 16):
                s = pl.ds(j * 16, 16)
                o_v[i, s] = jnp.maximum(g_v[i, s] + ew_v[i, s], 0.0)
            return cc
        lax.fori_loop(0, CB, eb, 0)

        pltpu.sync_copy(o_v, acc_sh.at[cidx], add=True)
        return c
    lax.fori_loop(0, CHUNKS, chunk, 0)

    plsc.subcore_barrier()

    for k in range(N_PER_TILE // ZR):
        r0 = sid * N_PER_TILE + k * ZR
        pltpu.sync_copy(acc_sh.at[pl.ds(r0, ZR)], z_v)
        pltpu.sync_copy(z_v, out_h.at[cid, pl.ds(r0, ZR)])


def _edge_stage(row, col, ew, xw):
    mesh = plsc.VectorSubcoreMesh(core_axis_name="c", subcore_axis_name="s",
                                  num_cores=NC, num_subcores=NS)
    f = pl.kernel(
        _edge_sc_body,
        out_type=jax.ShapeDtypeStruct((NC, NP, HID), jnp.float32),
        mesh=mesh,
        compiler_params=pltpu.CompilerParams(use_tc_tiling_on_sc=False),
        scratch_types=[
            pltpu.VMEM((CB,), jnp.int32),
            pltpu.VMEM((CB,), jnp.int32),
            pltpu.VMEM((CB, FH), jnp.float32),
            pltpu.VMEM((CB, FH), jnp.float32),
            pltpu.VMEM((CB, HID), jnp.float32),
            pltpu.VMEM((ZR, HID), jnp.float32),
            pltpu.VMEM_SHARED((NP, HID), jnp.float32),
            pltpu.SemaphoreType.DMA,
        ],
    )
    return f(row, col, ew, xw)


def _node_body(x_ref, p_ref, b_ref, u_ref, w1b_ref, b1b_ref,
               w2x_ref, w2m_ref, w2u_ref, b2a_ref, w2b_ref, b2b_ref, o_ref):
    s = jnp.concatenate([p_ref[0, :, :FH], p_ref[1, :, :FH]], axis=1)
    cnt = jnp.sum(p_ref[0, :, FH:], axis=1)
    safe = jnp.maximum(cnt, 1.0)
    mr = s / safe[:, None]
    meanh = (jnp.dot(mr, w1b_ref[...], preferred_element_type=jnp.float32)
             + b1b_ref[...] * (cnt > 0.0).astype(jnp.float32)[:, None])

    bidx = b_ref[0, 0, :]
    oh = (bidx[:, None] ==
          lax.broadcasted_iota(jnp.int32, (bidx.shape[0], G), 1)
          ).astype(jnp.float32)
    ub = jnp.dot(oh, u_ref[...], preferred_element_type=jnp.float32)

    a2 = (jnp.dot(x_ref[...], w2x_ref[...], preferred_element_type=jnp.float32)
          + jnp.dot(meanh, w2m_ref[...], preferred_element_type=jnp.float32)
          + jnp.dot(ub, w2u_ref[...], preferred_element_type=jnp.float32)
          + b2a_ref[...])
    o_ref[...] = (jnp.dot(jax.nn.relu(a2), w2b_ref[...],
                          preferred_element_type=jnp.float32) + b2b_ref[...])


def _node_stage(x, acc, batch3, u, W1b, b1b, W2a, b2a, W2b, b2b):
    bn = 1000
    full = lambda r, c: pl.BlockSpec((r, c), lambda i: (0, 0))
    return pl.pallas_call(
        _node_body,
        grid=(N // bn,),
        in_specs=[
            pl.BlockSpec((bn, N_F), lambda i: (i, 0)),
            pl.BlockSpec((NC, bn, HID), lambda i: (0, i, 0)),
            pl.BlockSpec((1, 1, bn), lambda i: (i, 0, 0)),
            full(G, U_F),
            full(HID, HID),
            full(1, HID),
            full(N_F, HID),
            full(HID, HID),
            full(U_F, HID),
            full(1, HID),
            full(HID, N_F),
            full(1, N_F),
        ],
        out_specs=pl.BlockSpec((bn, N_F), lambda i: (i, 0)),
        out_shape=jax.ShapeDtypeStruct((N, N_F), jnp.float32),
    )(x, acc, batch3, u, W1b, b1b.reshape(1, HID),
      W2a[:N_F], W2a[N_F:N_F + HID], W2a[N_F + HID:], b2a.reshape(1, HID),
      W2b, b2b.reshape(1, N_F))


def kernel(x, edge_index, edge_attr, u, batch,
           W1a, b1a, W1b, b1b, W2a, b2a, W2b, b2b):
    row = edge_index[0].astype(jnp.int32)
    col = edge_index[1].astype(jnp.int32)
    xw = _project_x(x, W1a[:N_F])
    ew = _project_edges(edge_attr, W1a[N_F:], b1a)
    acc = _edge_stage(row, col, ew, xw)
    batch3 = batch.astype(jnp.int32).reshape(N // 1000, 1, 1000)
    return _node_stage(x, acc, batch3, u, W1b, b1b, W2a, b2a, W2b, b2b)


# speedup vs baseline: 1.8690x; 1.3699x over previous
import jax
import jax.numpy as jnp
from jax import lax
from jax.experimental import pallas as pl
from jax.experimental.pallas import tpu as pltpu
from jax.experimental.pallas import tpu_sc as plsc

N = 10000
E = 320000
N_F = 128
E_F = 16
HID = 128
U_F = 16
G = 64

NC, NS = 2, 16
FH = HID // NC
E_PER_TILE = E // NS
CB = 80
CHUNKS = E_PER_TILE // CB
NP = 10240
ZR = 128
N_PER_TILE = NP // NS


def _xw_body(x_ref, w_ref, o_ref):
    o_ref[...] = jnp.dot(x_ref[...], w_ref[0],
                         preferred_element_type=jnp.float32)


def _project_x(x, W1a_x):
    bn = 2000
    return pl.pallas_call(
        _xw_body,
        grid=(NC, N // bn),
        in_specs=[
            pl.BlockSpec((bn, N_F), lambda c, i: (i, 0)),
            pl.BlockSpec((1, N_F, FH), lambda c, i: (c, 0, 0)),
        ],
        out_specs=pl.BlockSpec((bn, FH), lambda c, i: (c * (N // bn) + i, 0)),
        out_shape=jax.ShapeDtypeStruct((NC * N, FH), jnp.float32),
    )(x, W1a_x.reshape(N_F, NC, FH).transpose(1, 0, 2))


def _ew_body(ea_ref, w_ref, b_ref, o_ref):
    o_ref[...] = jnp.dot(ea_ref[...], w_ref[0],
                         preferred_element_type=jnp.float32) + b_ref[0]


def _project_edges(edge_attr, W1a_e, b1a):
    be = 8000
    return pl.pallas_call(
        _ew_body,
        grid=(NC, E // be),
        in_specs=[
            pl.BlockSpec((be, E_F), lambda c, i: (i, 0)),
            pl.BlockSpec((1, E_F, FH), lambda c, i: (c, 0, 0)),
            pl.BlockSpec((1, 1, FH), lambda c, i: (c, 0, 0)),
        ],
        out_specs=pl.BlockSpec((be, FH), lambda c, i: (c * (E // be) + i, 0)),
        out_shape=jax.ShapeDtypeStruct((NC * E, FH), jnp.float32),
    )(edge_attr, W1a_e.reshape(E_F, NC, FH).transpose(1, 0, 2),
      b1a.reshape(NC, 1, FH))


def _edge_sc_body(row_h, col_h, ew_h, xw_h, out_h,
                  ridx0, cidx0, sidx0, ew0, g0, o0,
                  ridx1, cidx1, sidx1, ew1, g1, o1,
                  acc_sh,
                  semL0, semL1, semG0, semG1, semS0, semS1):
    cid = lax.axis_index("c")
    sid = lax.axis_index("s")

    ridx = (ridx0, ridx1)
    cidx = (cidx0, cidx1)
    sidx = (sidx0, sidx1)
    ew_v = (ew0, ew1)
    g_v = (g0, g1)
    o_v = (o0, o1)
    semL = (semL0, semL1)
    semG = (semG0, semG1)
    semS = (semS0, semS1)

    zvec = jnp.zeros((16,), jnp.float32)
    cvec = jnp.where(lax.iota(jnp.int32, 16) == 0, 1.0, 0.0)

    def zb(i, c):
        for j in range(HID // 16):
            o0[i, pl.ds(j * 16, 16)] = zvec
        return c
    lax.fori_loop(0, CB, zb, 0)
    for k in range(N_PER_TILE // CB):
        pltpu.sync_copy(o0, acc_sh.at[pl.ds(sid * N_PER_TILE + k * CB, CB)])

    def cb(i, c):
        for b in range(2):
            o_v[b][i, pl.ds(FH, 16)] = cvec
            for j in range(FH // 16 + 1, HID // 16):
                o_v[b][i, pl.ds(j * 16, 16)] = zvec
        return c
    lax.fori_loop(0, CB, cb, 0)

    plsc.subcore_barrier()

    tile_base = sid * E_PER_TILE
    eoff = cid * E

    def issue_loads(t, b):
        base = tile_base + t * CB
        pltpu.async_copy(row_h.at[pl.ds(eoff + base, CB)], ridx[b], semL[b])
        pltpu.async_copy(col_h.at[pl.ds(base, CB)], cidx[b], semL[b])
        pltpu.async_copy(ew_h.at[pl.ds(eoff + base, CB)], ew_v[b], semL[b])

    def wait_loads(b):
        pltpu.make_async_copy(row_h.at[pl.ds(0, CB)], ridx[b], semL[b]).wait()
        pltpu.make_async_copy(col_h.at[pl.ds(0, CB)], cidx[b], semL[b]).wait()
        pltpu.make_async_copy(ew_h.at[pl.ds(0, CB)], ew_v[b], semL[b]).wait()

    def issue_gather(b):
        pltpu.async_copy(xw_h.at[ridx[b]], g_v[b], semG[b])

    def wait_gather(b):
        pltpu.make_async_copy(ew_h.at[pl.ds(0, CB)], g_v[b], semG[b]).wait()

    def issue_scatter(b):
        pltpu.async_copy(o_v[b], acc_sh.at[sidx[b]], semS[b], add=True)

    def wait_scatter(b):
        pltpu.make_async_copy(o_v[b], acc_sh.at[pl.ds(0, CB)], semS[b]).wait()

    issue_loads(0, 0)
    wait_loads(0)
    issue_gather(0)
    issue_loads(1, 1)

    def phase(t, b):
        wait_gather(b)

        @pl.when(t >= 2)
        def _():
            wait_scatter(b)

        def eb(i, cc):
            for j in range(FH // 16):
                s = pl.ds(j * 16, 16)
                o_v[b][i, s] = jnp.maximum(g_v[b][i, s] + ew_v[b][i, s], 0.0)
            return cc
        lax.fori_loop(0, CB, eb, 0)

        def sc(j, cc):
            s = pl.ds(j * 16, 16)
            sidx[b][s] = cidx[b][s]
            return cc
        lax.fori_loop(0, CB // 16, sc, 0)

        issue_scatter(b)

        @pl.when(t + 2 < CHUNKS)
        def _():
            issue_loads(t + 2, b)

        @pl.when(t + 1 < CHUNKS)
        def _():
            wait_loads(1 - b)
            issue_gather(1 - b)

    def pair(p, c):
        phase(2 * p, 0)
        phase(2 * p + 1, 1)
        return c
    lax.fori_loop(0, CHUNKS // 2, pair, 0)

    wait_scatter(0)
    wait_scatter(1)

    plsc.subcore_barrier()

    for k in range(N_PER_TILE // CB):
        r0 = sid * N_PER_TILE + k * CB
        pltpu.sync_copy(acc_sh.at[pl.ds(r0, CB)], o0)
        pltpu.sync_copy(o0, out_h.at[cid, pl.ds(r0, CB)])


def _edge_stage(radj, col, ew, xw):
    mesh = plsc.VectorSubcoreMesh(core_axis_name="c", subcore_axis_name="s",
                                  num_cores=NC, num_subcores=NS)
    buf = [pltpu.VMEM((CB,), jnp.int32),
           pltpu.VMEM((CB,), jnp.int32),
           pltpu.VMEM((CB,), jnp.int32),
           pltpu.VMEM((CB, FH), jnp.float32),
           pltpu.VMEM((CB, FH), jnp.float32),
           pltpu.VMEM((CB, HID), jnp.float32)]
    f = pl.kernel(
        _edge_sc_body,
        out_type=jax.ShapeDtypeStruct((NC, NP, HID), jnp.float32),
        mesh=mesh,
        compiler_params=pltpu.CompilerParams(use_tc_tiling_on_sc=False),
        scratch_types=buf + buf + [
            pltpu.VMEM_SHARED((NP, HID), jnp.float32),
        ] + [pltpu.SemaphoreType.DMA] * 6,
    )
    return f(radj, col, ew, xw)


def _node_body(x_ref, p_ref, b_ref, u_ref, w1b_ref, b1b_ref,
               w2x_ref, w2m_ref, w2u_ref, b2a_ref, w2b_ref, b2b_ref, o_ref):
    s = jnp.concatenate([p_ref[0, :, :FH], p_ref[1, :, :FH]], axis=1)
    cnt = jnp.sum(p_ref[0, :, FH:], axis=1)
    safe = jnp.maximum(cnt, 1.0)
    mr = s / safe[:, None]
    meanh = (jnp.dot(mr, w1b_ref[...], preferred_element_type=jnp.float32)
             + b1b_ref[...] * (cnt > 0.0).astype(jnp.float32)[:, None])

    bidx = b_ref[0, 0, :]
    oh = (bidx[:, None] ==
          lax.broadcasted_iota(jnp.int32, (bidx.shape[0], G), 1)
          ).astype(jnp.float32)
    ub = jnp.dot(oh, u_ref[...], preferred_element_type=jnp.float32)

    a2 = (jnp.dot(x_ref[...], w2x_ref[...], preferred_element_type=jnp.float32)
          + jnp.dot(meanh, w2m_ref[...], preferred_element_type=jnp.float32)
          + jnp.dot(ub, w2u_ref[...], preferred_element_type=jnp.float32)
          + b2a_ref[...])
    o_ref[...] = (jnp.dot(jax.nn.relu(a2), w2b_ref[...],
                          preferred_element_type=jnp.float32) + b2b_ref[...])


def _node_stage(x, acc, batch3, u, W1b, b1b, W2a, b2a, W2b, b2b):
    bn = 1000
    full = lambda r, c: pl.BlockSpec((r, c), lambda i: (0, 0))
    return pl.pallas_call(
        _node_body,
        grid=(N // bn,),
        in_specs=[
            pl.BlockSpec((bn, N_F), lambda i: (i, 0)),
            pl.BlockSpec((NC, bn, HID), lambda i: (0, i, 0)),
            pl.BlockSpec((1, 1, bn), lambda i: (i, 0, 0)),
            full(G, U_F),
            full(HID, HID),
            full(1, HID),
            full(N_F, HID),
            full(HID, HID),
            full(U_F, HID),
            full(1, HID),
            full(HID, N_F),
            full(1, N_F),
        ],
        out_specs=pl.BlockSpec((bn, N_F), lambda i: (i, 0)),
        out_shape=jax.ShapeDtypeStruct((N, N_F), jnp.float32),
    )(x, acc, batch3, u, W1b, b1b.reshape(1, HID),
      W2a[:N_F], W2a[N_F:N_F + HID], W2a[N_F + HID:], b2a.reshape(1, HID),
      W2b, b2b.reshape(1, N_F))


def kernel(x, edge_index, edge_attr, u, batch,
           W1a, b1a, W1b, b1b, W2a, b2a, W2b, b2b):
    row = edge_index[0].astype(jnp.int32)
    col = edge_index[1].astype(jnp.int32)
    xw = _project_x(x, W1a[:N_F])
    ew = _project_edges(edge_attr, W1a[N_F:], b1a)
    radj = jnp.concatenate([row, row + N])
    acc = _edge_stage(radj, col, ew, xw)
    batch3 = batch.astype(jnp.int32).reshape(N // 1000, 1, 1000)
    return _node_stage(x, acc, batch3, u, W1b, b1b, W2a, b2a, W2b, b2b)


# speedup vs baseline: 2.0805x; 1.1131x over previous
import jax
import jax.numpy as jnp
from jax import lax
from jax.experimental import pallas as pl
from jax.experimental.pallas import tpu as pltpu
from jax.experimental.pallas import tpu_sc as plsc

N = 10000
E = 320000
N_F = 128
E_F = 16
HID = 128
U_F = 16
G = 64

NC, NS = 2, 16
FH = HID // NC
E_PER_TILE = E // NS
CB = 80
CHUNKS = E_PER_TILE // CB
NP = 10240
ZR = 128
N_PER_TILE = NP // NS


def _xw_body(x_ref, w_ref, o_ref):
    o_ref[...] = jnp.dot(x_ref[...], w_ref[0],
                         preferred_element_type=jnp.float32)


def _project_x(x, W1a_x):
    bn = 2000
    return pl.pallas_call(
        _xw_body,
        grid=(NC, N // bn),
        in_specs=[
            pl.BlockSpec((bn, N_F), lambda c, i: (i, 0)),
            pl.BlockSpec((1, N_F, FH), lambda c, i: (c, 0, 0)),
        ],
        out_specs=pl.BlockSpec((bn, FH), lambda c, i: (c * (N // bn) + i, 0)),
        out_shape=jax.ShapeDtypeStruct((NC * N, FH), jnp.float32),
    )(x, W1a_x.reshape(N_F, NC, FH).transpose(1, 0, 2))


def _ew_body(ea_ref, w_ref, b_ref, o_ref):
    o_ref[...] = lax.dot_general(
        ea_ref[...], w_ref[0], (((0,), (0,)), ((), ())),
        preferred_element_type=jnp.float32) + b_ref[0]


def _project_edges(edge_attr, W1a_e, b1a):
    be = 6400
    return pl.pallas_call(
        _ew_body,
        grid=(NC, E // be),
        in_specs=[
            pl.BlockSpec((E_F, be), lambda c, i: (0, i)),
            pl.BlockSpec((1, E_F, FH), lambda c, i: (c, 0, 0)),
            pl.BlockSpec((1, 1, FH), lambda c, i: (c, 0, 0)),
        ],
        out_specs=pl.BlockSpec((be, FH), lambda c, i: (c * (E // be) + i, 0)),
        out_shape=jax.ShapeDtypeStruct((NC * E, FH), jnp.float32),
    )(edge_attr.T, W1a_e.reshape(E_F, NC, FH).transpose(1, 0, 2),
      b1a.reshape(NC, 1, FH))


def _edge_sc_body(row_h, col_h, ew_h, xw_h, out_h,
                  ridx0, cidx0, sidx0, ew0, g0, o0,
                  ridx1, cidx1, sidx1, ew1, g1, o1,
                  acc_sh,
                  semL0, semL1, semG0, semG1, semS0, semS1):
    cid = lax.axis_index("c")
    sid = lax.axis_index("s")

    ridx = (ridx0, ridx1)
    cidx = (cidx0, cidx1)
    sidx = (sidx0, sidx1)
    ew_v = (ew0, ew1)
    g_v = (g0, g1)
    o_v = (o0, o1)
    semL = (semL0, semL1)
    semG = (semG0, semG1)
    semS = (semS0, semS1)

    zvec = jnp.zeros((16,), jnp.float32)
    cvec = jnp.where(lax.iota(jnp.int32, 16) == 0, 1.0, 0.0)

    def zb(i, c):
        for j in range(HID // 16):
            o0[i, pl.ds(j * 16, 16)] = zvec
        return c
    lax.fori_loop(0, CB, zb, 0)
    for k in range(N_PER_TILE // CB):
        pltpu.sync_copy(o0, acc_sh.at[pl.ds(sid * N_PER_TILE + k * CB, CB)])

    def cb(i, c):
        for b in range(2):
            o_v[b][i, pl.ds(FH, 16)] = cvec
            for j in range(FH // 16 + 1, HID // 16):
                o_v[b][i, pl.ds(j * 16, 16)] = zvec
        return c
    lax.fori_loop(0, CB, cb, 0)

    plsc.subcore_barrier()

    tile_base = sid * E_PER_TILE
    eoff = cid * E

    def issue_loads(t, b):
        base = tile_base + t * CB
        pltpu.async_copy(row_h.at[pl.ds(eoff + base, CB)], ridx[b], semL[b])
        pltpu.async_copy(col_h.at[pl.ds(base, CB)], cidx[b], semL[b])
        pltpu.async_copy(ew_h.at[pl.ds(eoff + base, CB)], ew_v[b], semL[b])

    def wait_loads(b):
        pltpu.make_async_copy(row_h.at[pl.ds(0, CB)], ridx[b], semL[b]).wait()
        pltpu.make_async_copy(col_h.at[pl.ds(0, CB)], cidx[b], semL[b]).wait()
        pltpu.make_async_copy(ew_h.at[pl.ds(0, CB)], ew_v[b], semL[b]).wait()

    def issue_gather(b):
        pltpu.async_copy(xw_h.at[ridx[b]], g_v[b], semG[b])

    def wait_gather(b):
        pltpu.make_async_copy(ew_h.at[pl.ds(0, CB)], g_v[b], semG[b]).wait()

    def issue_scatter(b):
        pltpu.async_copy(o_v[b], acc_sh.at[sidx[b]], semS[b], add=True)

    def wait_scatter(b):
        pltpu.make_async_copy(o_v[b], acc_sh.at[pl.ds(0, CB)], semS[b]).wait()

    issue_loads(0, 0)
    wait_loads(0)
    issue_gather(0)
    issue_loads(1, 1)

    def phase(t, b):
        wait_gather(b)

        @pl.when(t >= 2)
        def _():
            wait_scatter(b)

        def eb(i, cc):
            for j in range(FH // 16):
                s = pl.ds(j * 16, 16)
                o_v[b][i, s] = jnp.maximum(g_v[b][i, s] + ew_v[b][i, s], 0.0)
            return cc
        lax.fori_loop(0, CB, eb, 0)

        def sc(j, cc):
            s = pl.ds(j * 16, 16)
            sidx[b][s] = cidx[b][s]
            return cc
        lax.fori_loop(0, CB // 16, sc, 0)

        issue_scatter(b)

        @pl.when(t + 2 < CHUNKS)
        def _():
            issue_loads(t + 2, b)

        @pl.when(t + 1 < CHUNKS)
        def _():
            wait_loads(1 - b)
            issue_gather(1 - b)

    def pair(p, c):
        phase(2 * p, 0)
        phase(2 * p + 1, 1)
        return c
    lax.fori_loop(0, CHUNKS // 2, pair, 0)

    wait_scatter(0)
    wait_scatter(1)

    plsc.subcore_barrier()

    for k in range(N_PER_TILE // CB):
        r0 = sid * N_PER_TILE + k * CB
        pltpu.sync_copy(acc_sh.at[pl.ds(r0, CB)], o0)
        pltpu.sync_copy(o0, out_h.at[cid, pl.ds(r0, CB)])


def _edge_stage(radj, col, ew, xw):
    mesh = plsc.VectorSubcoreMesh(core_axis_name="c", subcore_axis_name="s",
                                  num_cores=NC, num_subcores=NS)
    buf = [pltpu.VMEM((CB,), jnp.int32),
           pltpu.VMEM((CB,), jnp.int32),
           pltpu.VMEM((CB,), jnp.int32),
           pltpu.VMEM((CB, FH), jnp.float32),
           pltpu.VMEM((CB, FH), jnp.float32),
           pltpu.VMEM((CB, HID), jnp.float32)]
    f = pl.kernel(
        _edge_sc_body,
        out_type=jax.ShapeDtypeStruct((NC, NP, HID), jnp.float32),
        mesh=mesh,
        compiler_params=pltpu.CompilerParams(use_tc_tiling_on_sc=False),
        scratch_types=buf + buf + [
            pltpu.VMEM_SHARED((NP, HID), jnp.float32),
        ] + [pltpu.SemaphoreType.DMA] * 6,
    )
    return f(radj, col, ew, xw)


def _node_body(x_ref, p_ref, b_ref, u_ref, w1b_ref, b1b_ref,
               w2x_ref, w2m_ref, w2u_ref, b2a_ref, w2b_ref, b2b_ref, o_ref):
    s = jnp.concatenate([p_ref[0, :, :FH], p_ref[1, :, :FH]], axis=1)
    cnt = jnp.sum(p_ref[0, :, FH:], axis=1)
    safe = jnp.maximum(cnt, 1.0)
    mr = s / safe[:, None]
    meanh = (jnp.dot(mr, w1b_ref[...], preferred_element_type=jnp.float32)
             + b1b_ref[...] * (cnt > 0.0).astype(jnp.float32)[:, None])

    bidx = b_ref[0, 0, :]
    oh = (bidx[:, None] ==
          lax.broadcasted_iota(jnp.int32, (bidx.shape[0], G), 1)
          ).astype(jnp.float32)
    ub = jnp.dot(oh, u_ref[...], preferred_element_type=jnp.float32)

    a2 = (jnp.dot(x_ref[...], w2x_ref[...], preferred_element_type=jnp.float32)
          + jnp.dot(meanh, w2m_ref[...], preferred_element_type=jnp.float32)
          + jnp.dot(ub, w2u_ref[...], preferred_element_type=jnp.float32)
          + b2a_ref[...])
    o_ref[...] = (jnp.dot(jax.nn.relu(a2), w2b_ref[...],
                          preferred_element_type=jnp.float32) + b2b_ref[...])


def _node_stage(x, acc, batch3, u, W1b, b1b, W2a, b2a, W2b, b2b):
    bn = 1000
    full = lambda r, c: pl.BlockSpec((r, c), lambda i: (0, 0))
    return pl.pallas_call(
        _node_body,
        grid=(N // bn,),
        in_specs=[
            pl.BlockSpec((bn, N_F), lambda i: (i, 0)),
            pl.BlockSpec((NC, bn, HID), lambda i: (0, i, 0)),
            pl.BlockSpec((1, 1, bn), lambda i: (i, 0, 0)),
            full(G, U_F),
            full(HID, HID),
            full(1, HID),
            full(N_F, HID),
            full(HID, HID),
            full(U_F, HID),
            full(1, HID),
            full(HID, N_F),
            full(1, N_F),
        ],
        out_specs=pl.BlockSpec((bn, N_F), lambda i: (i, 0)),
        out_shape=jax.ShapeDtypeStruct((N, N_F), jnp.float32),
    )(x, acc, batch3, u, W1b, b1b.reshape(1, HID),
      W2a[:N_F], W2a[N_F:N_F + HID], W2a[N_F + HID:], b2a.reshape(1, HID),
      W2b, b2b.reshape(1, N_F))


def kernel(x, edge_index, edge_attr, u, batch,
           W1a, b1a, W1b, b1b, W2a, b2a, W2b, b2b):
    row = edge_index[0].astype(jnp.int32)
    col = edge_index[1].astype(jnp.int32)
    xw = _project_x(x, W1a[:N_F])
    ew = _project_edges(edge_attr, W1a[N_F:], b1a)
    radj = jnp.concatenate([row, row + N])
    acc = _edge_stage(radj, col, ew, xw)
    batch3 = batch.astype(jnp.int32).reshape(N // 1000, 1, 1000)
    return _node_stage(x, acc, batch3, u, W1b, b1b, W2a, b2a, W2b, b2b)


# speedup vs baseline: 2.1204x; 1.0192x over previous
import jax
import jax.numpy as jnp
from jax import lax
from jax.experimental import pallas as pl
from jax.experimental.pallas import tpu as pltpu
from jax.experimental.pallas import tpu_sc as plsc

N = 10000
E = 320000
N_F = 128
E_F = 16
HID = 128
U_F = 16
G = 64

NC, NS = 2, 16
FH = HID // NC
E_PER_TILE = E // NS
CB = 80
CHUNKS = E_PER_TILE // CB
NP = 10240
ZR = 128
N_PER_TILE = NP // NS


def _xw_body(x_ref, w_ref, o_ref):
    o_ref[...] = jnp.dot(x_ref[...], w_ref[0],
                         preferred_element_type=jnp.float32)


def _project_x(x, W1a_x):
    bn = 2000
    return pl.pallas_call(
        _xw_body,
        grid=(NC, N // bn),
        in_specs=[
            pl.BlockSpec((bn, N_F), lambda c, i: (i, 0)),
            pl.BlockSpec((1, N_F, FH), lambda c, i: (c, 0, 0)),
        ],
        out_specs=pl.BlockSpec((bn, FH), lambda c, i: (c * (N // bn) + i, 0)),
        out_shape=jax.ShapeDtypeStruct((NC * N, FH), jnp.float32),
    )(x, W1a_x.reshape(N_F, NC, FH).transpose(1, 0, 2))


def _ew_body(ea_ref, w_ref, b_ref, o_ref):
    o_ref[...] = lax.dot_general(
        ea_ref[...], w_ref[0], (((0,), (0,)), ((), ())),
        preferred_element_type=jnp.float32) + b_ref[0]


def _project_edges(edge_attr, W1a_e, b1a):
    be = 6400
    return pl.pallas_call(
        _ew_body,
        grid=(NC, E // be),
        in_specs=[
            pl.BlockSpec((E_F, be), lambda c, i: (0, i)),
            pl.BlockSpec((1, E_F, FH), lambda c, i: (c, 0, 0)),
            pl.BlockSpec((1, 1, FH), lambda c, i: (c, 0, 0)),
        ],
        out_specs=pl.BlockSpec((be, FH), lambda c, i: (c * (E // be) + i, 0)),
        out_shape=jax.ShapeDtypeStruct((NC * E, FH), jnp.float32),
    )(edge_attr.T, W1a_e.reshape(E_F, NC, FH).transpose(1, 0, 2),
      b1a.reshape(NC, 1, FH))


def _edge_sc_body(row_h, col_h, ew_h, xw_h, out_h,
                  ridx0, cidx0, sidx0, ew0, g0, o0,
                  ridx1, cidx1, sidx1, ew1, g1, o1,
                  acc_sh,
                  semL0, semL1, semG0, semG1, semS0, semS1):
    cid = lax.axis_index("c")
    sid = lax.axis_index("s")

    ridx = (ridx0, ridx1)
    cidx = (cidx0, cidx1)
    sidx = (sidx0, sidx1)
    ew_v = (ew0, ew1)
    g_v = (g0, g1)
    o_v = (o0, o1)
    semL = (semL0, semL1)
    semG = (semG0, semG1)
    semS = (semS0, semS1)

    zvec = jnp.zeros((16,), jnp.float32)
    cvec = jnp.where(lax.iota(jnp.int32, 16) == 0, 1.0, 0.0)

    def zb(i, c):
        for j in range(HID // 16):
            o0[i, pl.ds(j * 16, 16)] = zvec
        return c
    lax.fori_loop(0, CB, zb, 0)
    for k in range(N_PER_TILE // CB):
        pltpu.sync_copy(o0, acc_sh.at[pl.ds(sid * N_PER_TILE + k * CB, CB)])

    def cb(i, c):
        for b in range(2):
            o_v[b][i, pl.ds(FH, 16)] = cvec
            for j in range(FH // 16 + 1, HID // 16):
                o_v[b][i, pl.ds(j * 16, 16)] = zvec
        return c
    lax.fori_loop(0, CB, cb, 0)

    plsc.subcore_barrier()

    tile_base = sid * E_PER_TILE
    eoff = cid * E

    def issue_loads(t, b):
        base = tile_base + t * CB
        pltpu.async_copy(row_h.at[pl.ds(eoff + base, CB)], ridx[b], semL[b])
        pltpu.async_copy(col_h.at[pl.ds(base, CB)], cidx[b], semL[b])
        pltpu.async_copy(ew_h.at[pl.ds(eoff + base, CB)], ew_v[b], semL[b])

    def wait_loads(b):
        pltpu.make_async_copy(row_h.at[pl.ds(0, CB)], ridx[b], semL[b]).wait()
        pltpu.make_async_copy(col_h.at[pl.ds(0, CB)], cidx[b], semL[b]).wait()
        pltpu.make_async_copy(ew_h.at[pl.ds(0, CB)], ew_v[b], semL[b]).wait()

    def issue_gather(b):
        pltpu.async_copy(xw_h.at[ridx[b]], g_v[b], semG[b])

    def wait_gather(b):
        pltpu.make_async_copy(ew_h.at[pl.ds(0, CB)], g_v[b], semG[b]).wait()

    def issue_scatter(b):
        pltpu.async_copy(o_v[b], acc_sh.at[sidx[b]], semS[b], add=True)

    def wait_scatter(b):
        pltpu.make_async_copy(o_v[b], acc_sh.at[pl.ds(0, CB)], semS[b]).wait()

    issue_loads(0, 0)
    wait_loads(0)
    issue_gather(0)
    issue_loads(1, 1)

    def phase(t, b):
        wait_gather(b)

        @pl.when(t + 1 < CHUNKS)
        def _():
            wait_loads(1 - b)
            issue_gather(1 - b)

        @pl.when(t >= 2)
        def _():
            wait_scatter(b)

        def eb(i, cc):
            for j in range(FH // 16):
                s = pl.ds(j * 16, 16)
                o_v[b][i, s] = jnp.maximum(g_v[b][i, s] + ew_v[b][i, s], 0.0)
            return cc
        lax.fori_loop(0, CB, eb, 0, unroll=4)

        def sc(j, cc):
            s = pl.ds(j * 16, 16)
            sidx[b][s] = cidx[b][s]
            return cc
        lax.fori_loop(0, CB // 16, sc, 0, unroll=5)

        issue_scatter(b)

        @pl.when(t + 2 < CHUNKS)
        def _():
            issue_loads(t + 2, b)

    def pair(p, c):
        phase(2 * p, 0)
        phase(2 * p + 1, 1)
        return c
    lax.fori_loop(0, CHUNKS // 2, pair, 0)

    wait_scatter(0)
    wait_scatter(1)

    plsc.subcore_barrier()

    for k in range(N_PER_TILE // CB):
        r0 = sid * N_PER_TILE + k * CB
        pltpu.sync_copy(acc_sh.at[pl.ds(r0, CB)], o0)
        pltpu.sync_copy(o0, out_h.at[cid, pl.ds(r0, CB)])


def _edge_stage(radj, col, ew, xw):
    mesh = plsc.VectorSubcoreMesh(core_axis_name="c", subcore_axis_name="s",
                                  num_cores=NC, num_subcores=NS)
    buf = [pltpu.VMEM((CB,), jnp.int32),
           pltpu.VMEM((CB,), jnp.int32),
           pltpu.VMEM((CB,), jnp.int32),
           pltpu.VMEM((CB, FH), jnp.float32),
           pltpu.VMEM((CB, FH), jnp.float32),
           pltpu.VMEM((CB, HID), jnp.float32)]
    f = pl.kernel(
        _edge_sc_body,
        out_type=jax.ShapeDtypeStruct((NC, NP, HID), jnp.float32),
        mesh=mesh,
        compiler_params=pltpu.CompilerParams(use_tc_tiling_on_sc=False),
        scratch_types=buf + buf + [
            pltpu.VMEM_SHARED((NP, HID), jnp.float32),
        ] + [pltpu.SemaphoreType.DMA] * 6,
    )
    return f(radj, col, ew, xw)


def _node_body(x_ref, p_ref, b_ref, u_ref, w1b_ref, b1b_ref,
               w2x_ref, w2m_ref, w2u_ref, b2a_ref, w2b_ref, b2b_ref, o_ref):
    s = jnp.concatenate([p_ref[0, :, :FH], p_ref[1, :, :FH]], axis=1)
    cnt = jnp.sum(p_ref[0, :, FH:], axis=1)
    safe = jnp.maximum(cnt, 1.0)
    mr = s / safe[:, None]
    meanh = (jnp.dot(mr, w1b_ref[...], preferred_element_type=jnp.float32)
             + b1b_ref[...] * (cnt > 0.0).astype(jnp.float32)[:, None])

    bidx = b_ref[0, 0, :]
    oh = (bidx[:, None] ==
          lax.broadcasted_iota(jnp.int32, (bidx.shape[0], G), 1)
          ).astype(jnp.float32)
    ub = jnp.dot(oh, u_ref[...], preferred_element_type=jnp.float32)

    a2 = (jnp.dot(x_ref[...], w2x_ref[...], preferred_element_type=jnp.float32)
          + jnp.dot(meanh, w2m_ref[...], preferred_element_type=jnp.float32)
          + jnp.dot(ub, w2u_ref[...], preferred_element_type=jnp.float32)
          + b2a_ref[...])
    o_ref[...] = (jnp.dot(jax.nn.relu(a2), w2b_ref[...],
                          preferred_element_type=jnp.float32) + b2b_ref[...])


def _node_stage(x, acc, batch3, u, W1b, b1b, W2a, b2a, W2b, b2b):
    bn = 1000
    full = lambda r, c: pl.BlockSpec((r, c), lambda i: (0, 0))
    return pl.pallas_call(
        _node_body,
        grid=(N // bn,),
        in_specs=[
            pl.BlockSpec((bn, N_F), lambda i: (i, 0)),
            pl.BlockSpec((NC, bn, HID), lambda i: (0, i, 0)),
            pl.BlockSpec((1, 1, bn), lambda i: (i, 0, 0)),
            full(G, U_F),
            full(HID, HID),
            full(1, HID),
            full(N_F, HID),
            full(HID, HID),
            full(U_F, HID),
            full(1, HID),
            full(HID, N_F),
            full(1, N_F),
        ],
        out_specs=pl.BlockSpec((bn, N_F), lambda i: (i, 0)),
        out_shape=jax.ShapeDtypeStruct((N, N_F), jnp.float32),
    )(x, acc, batch3, u, W1b, b1b.reshape(1, HID),
      W2a[:N_F], W2a[N_F:N_F + HID], W2a[N_F + HID:], b2a.reshape(1, HID),
      W2b, b2b.reshape(1, N_F))


def kernel(x, edge_index, edge_attr, u, batch,
           W1a, b1a, W1b, b1b, W2a, b2a, W2b, b2b):
    row = edge_index[0].astype(jnp.int32)
    col = edge_index[1].astype(jnp.int32)
    xw = _project_x(x, W1a[:N_F])
    ew = _project_edges(edge_attr, W1a[N_F:], b1a)
    radj = jnp.concatenate([row, row + N])
    acc = _edge_stage(radj, col, ew, xw)
    batch3 = batch.astype(jnp.int32).reshape(N // 1000, 1, 1000)
    return _node_stage(x, acc, batch3, u, W1b, b1b, W2a, b2a, W2b, b2b)
